# Initial kernel scaffold; baseline (speedup 1.0000x reference)
import functools

import jax
import jax.numpy as jnp
from jax import lax
from jax.experimental import pallas as pl
from jax.experimental.pallas import tpu as pltpu
from jax.experimental.pallas import tpu_sc as plsc

NUM_CORES = 2
NUM_SUBCORES = 16
NW = NUM_CORES * NUM_SUBCORES


def _sc_gather(table, idx, chunk):
    n = idx.shape[0]
    d = table.shape[1]
    b_per_w = n // NW
    nch = b_per_w // chunk
    assert b_per_w * NW == n and nch * chunk == b_per_w and chunk % 8 == 0

    mesh = plsc.VectorSubcoreMesh(core_axis_name="c", subcore_axis_name="s")

    @functools.partial(
        pl.kernel,
        mesh=mesh,
        out_type=jax.ShapeDtypeStruct((n, d), jnp.float32),
        scratch_types=[
            pltpu.VMEM((b_per_w,), jnp.int32),
            pltpu.VMEM((chunk, d), jnp.float32),
            pltpu.VMEM((chunk, d), jnp.float32),
            pltpu.SemaphoreType.DMA,
            pltpu.SemaphoreType.DMA,
            pltpu.SemaphoreType.DMA,
            pltpu.SemaphoreType.DMA,
        ],
    )
    def gather_kernel(table_hbm, idx_hbm, out_hbm, idx_v, buf0, buf1,
                      gs0, gs1, os0, os1):
        wid = lax.axis_index("s") * NUM_CORES + lax.axis_index("c")
        base = wid * b_per_w
        pltpu.sync_copy(idx_hbm.at[pl.ds(base, b_per_w)], idx_v)

        bufs = (buf0, buf1)
        gsems = (gs0, gs1)
        osems = (os0, os1)

        def start_gather(c, buf, sem):
            return pltpu.async_copy(
                table_hbm.at[idx_v.at[pl.ds(c * chunk, chunk)]], buf, sem)

        gcp = [start_gather(0, bufs[0], gsems[0]), None]
        ocp = [None, None]
        for c in range(nch):
            cur = c & 1
            nxt = 1 - cur
            if c + 1 < nch:
                if ocp[nxt] is not None:
                    ocp[nxt].wait()
                gcp[nxt] = start_gather(c + 1, bufs[nxt], gsems[nxt])
            gcp[cur].wait()
            ocp[cur] = pltpu.async_copy(
                bufs[cur], out_hbm.at[pl.ds(base + c * chunk, chunk)],
                osems[cur])
        for cp in ocp:
            if cp is not None:
                cp.wait()

    return gather_kernel(table, idx)


def _tc_project(emb, B_w, B_b, blk):
    m, r = emb.shape
    e = B_w.shape[0]
    assert m % blk == 0
    bias = B_b.reshape(1, e)

    def proj_kernel(emb_ref, w_ref, b_ref, out_ref):
        out_ref[...] = (
            jnp.dot(emb_ref[...], w_ref[...].T,
                    preferred_element_type=jnp.float32,
                    precision=lax.Precision.HIGHEST)
            + b_ref[...]
        )

    return pl.pallas_call(
        proj_kernel,
        grid=(m // blk,),
        in_specs=[
            pl.BlockSpec((blk, r), lambda i: (i, 0)),
            pl.BlockSpec((e, r), lambda i: (0, 0)),
            pl.BlockSpec((1, e), lambda i: (0, 0)),
        ],
        out_specs=pl.BlockSpec((blk, e), lambda i: (i, 0)),
        out_shape=jax.ShapeDtypeStruct((m, e), jnp.float32),
        compiler_params=pltpu.CompilerParams(
            dimension_semantics=("arbitrary",)),
    )(emb, B_w, bias)


def kernel(x, A, B_w, B_b):
    batch, fields = x.shape
    embed = B_w.shape[0]
    n = batch * fields
    idx = x.reshape(n)
    emb = _sc_gather(A, idx, chunk=1664)
    out = _tc_project(emb, B_w, B_b, blk=4096)
    return out.reshape(batch, fields, embed)


# speedup vs baseline: 7.9090x; 7.9090x over previous
import functools

import jax
import jax.numpy as jnp
from jax import lax
from jax.experimental import pallas as pl
from jax.experimental.pallas import tpu as pltpu
from jax.experimental.pallas import tpu_sc as plsc

NUM_CORES = 2
NUM_SUBCORES = 16
NW = NUM_CORES * NUM_SUBCORES


def _sc_gather(table, idx, chunk):
    n = idx.shape[0]
    d = table.shape[1]
    b_per_w = n // NW
    nch = b_per_w // chunk
    assert b_per_w * NW == n and nch * chunk == b_per_w and chunk % 8 == 0

    mesh = plsc.VectorSubcoreMesh(core_axis_name="c", subcore_axis_name="s")

    @functools.partial(
        pl.kernel,
        mesh=mesh,
        out_type=jax.ShapeDtypeStruct((n, d), jnp.float32),
        scratch_types=[
            pltpu.VMEM((b_per_w,), jnp.int32),
            pltpu.VMEM((chunk, d), jnp.float32),
            pltpu.VMEM((chunk, d), jnp.float32),
            pltpu.SemaphoreType.DMA,
            pltpu.SemaphoreType.DMA,
            pltpu.SemaphoreType.DMA,
            pltpu.SemaphoreType.DMA,
        ],
        compiler_params=pltpu.CompilerParams(use_tc_tiling_on_sc=False),
    )
    def gather_kernel(table_hbm, idx_hbm, out_hbm, idx_v, buf0, buf1,
                      gs0, gs1, os0, os1):
        wid = lax.axis_index("s") * NUM_CORES + lax.axis_index("c")
        base = wid * b_per_w
        pltpu.sync_copy(idx_hbm.at[pl.ds(base, b_per_w)], idx_v)

        bufs = (buf0, buf1)
        gsems = (gs0, gs1)
        osems = (os0, os1)

        def start_gather(c, buf, sem):
            return pltpu.async_copy(
                table_hbm.at[idx_v.at[pl.ds(c * chunk, chunk)]], buf, sem)

        gcp = [start_gather(0, bufs[0], gsems[0]), None]
        ocp = [None, None]
        for c in range(nch):
            cur = c & 1
            nxt = 1 - cur
            if c + 1 < nch:
                if ocp[nxt] is not None:
                    ocp[nxt].wait()
                gcp[nxt] = start_gather(c + 1, bufs[nxt], gsems[nxt])
            gcp[cur].wait()
            ocp[cur] = pltpu.async_copy(
                bufs[cur], out_hbm.at[pl.ds(base + c * chunk, chunk)],
                osems[cur])
        for cp in ocp:
            if cp is not None:
                cp.wait()

    return gather_kernel(table, idx)


def _tc_project(emb, B_w, B_b, blk):
    m, r = emb.shape
    e = B_w.shape[0]
    assert m % blk == 0
    bias = B_b.reshape(1, e)

    def proj_kernel(emb_ref, w_ref, b_ref, out_ref):
        out_ref[...] = (
            jnp.dot(emb_ref[...], w_ref[...].T,
                    preferred_element_type=jnp.float32,
                    precision=lax.Precision.HIGHEST)
            + b_ref[...]
        )

    return pl.pallas_call(
        proj_kernel,
        grid=(m // blk,),
        in_specs=[
            pl.BlockSpec((blk, r), lambda i: (i, 0)),
            pl.BlockSpec((e, r), lambda i: (0, 0)),
            pl.BlockSpec((1, e), lambda i: (0, 0)),
        ],
        out_specs=pl.BlockSpec((blk, e), lambda i: (i, 0)),
        out_shape=jax.ShapeDtypeStruct((m, e), jnp.float32),
        compiler_params=pltpu.CompilerParams(
            dimension_semantics=("arbitrary",)),
    )(emb, B_w, bias)


def kernel(x, A, B_w, B_b):
    batch, fields = x.shape
    embed = B_w.shape[0]
    n = batch * fields
    idx = x.reshape(n)
    emb = _sc_gather(A, idx, chunk=1664)
    out = _tc_project(emb, B_w, B_b, blk=4096)
    return out.reshape(batch, fields, embed)


# speedup vs baseline: 10.6354x; 1.3447x over previous
import functools

import jax
import jax.numpy as jnp
from jax import lax
from jax.experimental import pallas as pl
from jax.experimental.pallas import tpu as pltpu
from jax.experimental.pallas import tpu_sc as plsc

NUM_CORES = 2
NUM_SUBCORES = 16
NW = NUM_CORES * NUM_SUBCORES
LANES = 128


def _sc_gather(table, idx, chunk):
    n = idx.shape[0]
    d = table.shape[1]
    b_per_w = n // NW
    nch = b_per_w // chunk
    assert b_per_w * NW == n and nch * chunk == b_per_w and chunk % 8 == 0

    mesh = plsc.VectorSubcoreMesh(core_axis_name="c", subcore_axis_name="s")

    @functools.partial(
        pl.kernel,
        mesh=mesh,
        out_type=jax.ShapeDtypeStruct((n, d), jnp.float32),
        scratch_types=[
            pltpu.VMEM((b_per_w,), jnp.int32),
            pltpu.VMEM((chunk, d), jnp.float32),
            pltpu.VMEM((chunk, d), jnp.float32),
            pltpu.SemaphoreType.DMA,
            pltpu.SemaphoreType.DMA,
            pltpu.SemaphoreType.DMA,
            pltpu.SemaphoreType.DMA,
        ],
        compiler_params=pltpu.CompilerParams(use_tc_tiling_on_sc=False),
    )
    def gather_kernel(table_hbm, idx_hbm, out_hbm, idx_v, buf0, buf1,
                      gs0, gs1, os0, os1):
        wid = lax.axis_index("s") * NUM_CORES + lax.axis_index("c")
        base = wid * b_per_w
        pltpu.sync_copy(idx_hbm.at[pl.ds(base, b_per_w)], idx_v)

        bufs = (buf0, buf1)
        gsems = (gs0, gs1)
        osems = (os0, os1)

        def start_gather(c, buf, sem):
            return pltpu.async_copy(
                table_hbm.at[idx_v.at[pl.ds(c * chunk, chunk)]], buf, sem)

        gcp = [start_gather(0, bufs[0], gsems[0]), None]
        ocp = [None, None]
        for c in range(nch):
            cur = c & 1
            nxt = 1 - cur
            if c + 1 < nch:
                if ocp[nxt] is not None:
                    ocp[nxt].wait()
                gcp[nxt] = start_gather(c + 1, bufs[nxt], gsems[nxt])
            gcp[cur].wait()
            ocp[cur] = pltpu.async_copy(
                bufs[cur], out_hbm.at[pl.ds(base + c * chunk, chunk)],
                osems[cur])
        for cp in ocp:
            if cp is not None:
                cp.wait()

    return gather_kernel(table, idx)


def _tc_project(emb128, G, bias512, blk):
    m8 = emb128.shape[0]
    nout = G.shape[1]
    assert m8 % blk == 0

    def proj_kernel(emb_ref, w_ref, b_ref, out_ref):
        out_ref[...] = (
            jnp.dot(emb_ref[...], w_ref[...],
                    preferred_element_type=jnp.float32,
                    precision=lax.Precision.HIGHEST)
            + b_ref[...]
        )

    return pl.pallas_call(
        proj_kernel,
        grid=(m8 // blk,),
        in_specs=[
            pl.BlockSpec((blk, LANES), lambda i: (i, 0)),
            pl.BlockSpec((LANES, nout), lambda i: (0, 0)),
            pl.BlockSpec((1, nout), lambda i: (0, 0)),
        ],
        out_specs=pl.BlockSpec((blk, nout), lambda i: (i, 0)),
        out_shape=jax.ShapeDtypeStruct((m8, nout), jnp.float32),
        compiler_params=pltpu.CompilerParams(
            dimension_semantics=("arbitrary",)),
    )(emb128, G, bias512)


def kernel(x, A, B_w, B_b):
    batch, fields = x.shape
    vocab, rank = A.shape
    embed = B_w.shape[0]
    n = batch * fields
    pack = LANES // rank

    idx = x.reshape(n)
    emb = _sc_gather(A, idx, chunk=1664)
    emb128 = emb.reshape(n * rank // LANES, LANES)

    G = jnp.kron(jnp.eye(pack, dtype=jnp.float32), B_w.T)
    bias512 = jnp.tile(B_b, pack).reshape(1, pack * embed)
    out = _tc_project(emb128, G, bias512, blk=1664)
    return out.reshape(batch, fields, embed)
